# baseline (speedup 1.0000x reference)
import jax
import jax.numpy as jnp
from jax.experimental import pallas as pl
from jax.experimental.pallas import tpu as pltpu

_HR = 2048
_LR = 512
_KP = 262144.0
_DIL = 15
_BR = 16
_EX = _BR + 14


def _a1_body(x_ref, w1_ref, b1_ref, wsc_ref, wp9_ref, g_ref, t9_ref):
    x = x_ref[...].reshape(27, 128 * _LR)
    f = jnp.maximum(
        jnp.dot(w1_ref[...], x, preferred_element_type=jnp.float32)
        + b1_ref[...], 0.0)
    g = jnp.dot(wsc_ref[...], f, preferred_element_type=jnp.float32)
    t9 = jnp.dot(wp9_ref[...], f, preferred_element_type=jnp.float32)
    g_ref[...] = g.reshape(32, 128, _LR)
    t9_ref[...] = t9.reshape(9, 128, _LR)


def _a2_body(t9_ref, bp_ref, p_ref):
    acc = jnp.zeros((_LR, _LR), jnp.float32) + bp_ref[0, 0]
    t9 = t9_ref[...]
    for ky in range(3):
        for kx in range(3):
            dy, dx = ky - 1, kx - 1
            a, b = max(0, -dy), _LR - max(0, dy)
            c, d = max(0, -dx), _LR - max(0, dx)
            sl = t9[ky * 3 + kx, a + dy:b + dy, c + dx:d + dx]
            acc = acc + jnp.pad(sl, ((a, _LR - b), (c, _LR - d)))
    p_ref[...] = jax.nn.sigmoid(acc)


def _b_body(p_ref, hr_ref, g_ref, w4_ref, bs1_ref, ws2_ref, bs2_ref,
            o_ref, carry_ref, nm_ref, sel_ref, ge_ref):
    i = pl.program_id(0)

    @pl.when(i == 0)
    def _():
        carry_ref[0, 0] = 0.0

    j = _BR * i - 7 + jax.lax.broadcasted_iota(jnp.int32, (_EX, 1), 0)
    jc = jnp.clip(j, 0, _HR - 1)
    f = (jc.astype(jnp.float32) + 0.5) * 0.25 - 0.5
    a = jnp.floor(f)
    w = f - a
    ai = a.astype(jnp.int32)
    a0 = jnp.clip(ai, 0, _LR - 1)
    a1 = jnp.clip(ai + 1, 0, _LR - 1)
    lanes = jax.lax.broadcasted_iota(jnp.int32, (_EX, _LR), 1)
    vmat = (jnp.where(lanes == a0, 1.0 - w, 0.0)
            + jnp.where(lanes == a1, w, 0.0))
    upv = jnp.dot(vmat, p_ref[...], preferred_element_type=jnp.float32)

    prv = jnp.concatenate([upv[:, :1], upv[:, :-1]], axis=1)
    nxt = jnp.concatenate([upv[:, 1:], upv[:, -1:]], axis=1)
    r4 = jax.lax.broadcasted_iota(jnp.int32, (1, _HR), 1) % 4
    wprv = jnp.where(r4 == 0, 0.375, jnp.where(r4 == 1, 0.125, 0.0))
    wcur = jnp.where((r4 == 0) | (r4 == 3), 0.625, 0.875)
    wnxt = jnp.where(r4 == 2, 0.125, jnp.where(r4 == 3, 0.375, 0.0))
    up_ext = (jnp.repeat(prv, 4, axis=1) * wprv
              + jnp.repeat(upv, 4, axis=1) * wcur
              + jnp.repeat(nxt, 4, axis=1) * wnxt)

    valid = ((j >= 0) & (j < _HR)).astype(jnp.float32)
    m_ext = ((up_ext > 0.01) & (up_ext < 0.99)).astype(jnp.float32) * valid

    dv = m_ext[0:_BR]
    for s in range(1, _DIL):
        dv = jnp.maximum(dv, m_ext[s:s + _BR])
    ph = jnp.concatenate(
        [jnp.zeros((_BR, 7), jnp.float32), dv, jnp.zeros((_BR, 7), jnp.float32)],
        axis=1)
    dil = ph[:, 0:_HR]
    for s in range(1, _DIL):
        dil = jnp.maximum(dil, ph[:, s:s + _HR])

    rowsum = jnp.sum(dil, axis=1, keepdims=True)
    q = jax.lax.broadcasted_iota(jnp.int32, (_BR, _BR), 0)
    r = jax.lax.broadcasted_iota(jnp.int32, (_BR, _BR), 1)
    row_excl = jnp.dot((r < q).astype(jnp.float32), rowsum,
                       preferred_element_type=jnp.float32)
    m3 = dil.reshape(_BR, 16, 128)
    csums = jnp.sum(m3, axis=-1)
    k16 = jax.lax.broadcasted_iota(jnp.int32, (16, 16), 0)
    c16 = jax.lax.broadcasted_iota(jnp.int32, (16, 16), 1)
    chunk_excl = jnp.dot(csums, (k16 < c16).astype(jnp.float32),
                         preferred_element_type=jnp.float32)
    i128 = jax.lax.broadcasted_iota(jnp.int32, (128, 128), 0)
    l128 = jax.lax.broadcasted_iota(jnp.int32, (128, 128), 1)
    within = jnp.dot(m3.reshape(_BR * 16, 128),
                     (i128 <= l128).astype(jnp.float32),
                     preferred_element_type=jnp.float32).reshape(_BR, 16, 128)
    incl = (within + chunk_excl[:, :, None] + row_excl[:, :, None]
            ).reshape(_BR, _HR) + carry_ref[0, 0]
    refined = (dil > 0.0) & (incl <= _KP)
    carry_ref[0, 0] += jnp.sum(rowsum)

    up_blk = up_ext[7:7 + _BR]
    w4 = w4_ref[...]
    ws2 = ws2_ref[...]
    bs1 = bs1_ref[...]
    bs2 = bs2_ref[0, 0]
    nm_ref[...] = (up_blk - 0.5) * 2.0
    sel_ref[...] = jnp.where(refined, 1.0, 0.0)
    g = g_ref[...].reshape(32, _BR // 4, _LR)
    ge_ref[...] = jnp.repeat(g, 4, axis=-1)
    o_ref[...] = up_blk

    def row_body(rr, _):
        hr_r = hr_ref[:, pl.ds(rr, 1), :].reshape(3, _HR)
        nm_r = nm_ref[pl.ds(rr, 1), :]
        x4_r = jnp.concatenate([hr_r, nm_r], axis=0)
        ge_r = ge_ref[:, pl.ds(rr // 4, 1), :].reshape(32, _HR)
        h1 = jnp.maximum(
            jnp.dot(w4, x4_r, preferred_element_type=jnp.float32)
            + ge_r + bs1, 0.0)
        o_r = jax.nn.sigmoid(
            jnp.dot(ws2, h1, preferred_element_type=jnp.float32) + bs2)
        keep = sel_ref[pl.ds(rr, 1), :] > 0.0
        o_ref[pl.ds(rr, 1), :] = jnp.where(keep, o_r, o_ref[pl.ds(rr, 1), :])
        return 0

    jax.lax.fori_loop(0, _BR, row_body, 0)


def kernel(lr_image, hr_image, pos, W1, b1, Wp, bp, Ws1, bs1, Ws2, bs2):
    lr = lr_image.reshape(3, _LR, _LR)
    hr = hr_image.reshape(3, _HR, _HR)

    lrp = jnp.pad(lr, ((0, 0), (1, 1), (1, 1)))
    x27 = jnp.stack(
        [lrp[c, ky:ky + _LR, kx:kx + _LR]
         for c in range(3) for ky in range(3) for kx in range(3)], axis=0)

    w1m = W1.reshape(32, 27)
    wp9 = jnp.transpose(Wp.reshape(32, 9))
    wsct = jnp.transpose(Ws1[4:])
    w4t = jnp.transpose(Ws1[:4])

    g3, t9 = pl.pallas_call(
        _a1_body,
        grid=(4,),
        in_specs=[
            pl.BlockSpec((27, 128, _LR), lambda i: (0, i, 0)),
            pl.BlockSpec((32, 27), lambda i: (0, 0)),
            pl.BlockSpec((32, 1), lambda i: (0, 0)),
            pl.BlockSpec((32, 32), lambda i: (0, 0)),
            pl.BlockSpec((9, 32), lambda i: (0, 0)),
        ],
        out_specs=[
            pl.BlockSpec((32, 128, _LR), lambda i: (0, i, 0)),
            pl.BlockSpec((9, 128, _LR), lambda i: (0, i, 0)),
        ],
        out_shape=[
            jax.ShapeDtypeStruct((32, _LR, _LR), jnp.float32),
            jax.ShapeDtypeStruct((9, _LR, _LR), jnp.float32),
        ],
    )(x27, w1m, b1.reshape(32, 1), wsct, wp9)

    p = pl.pallas_call(
        _a2_body,
        in_specs=[
            pl.BlockSpec((9, _LR, _LR), lambda: (0, 0, 0)),
            pl.BlockSpec((1, 1), lambda: (0, 0)),
        ],
        out_specs=pl.BlockSpec((_LR, _LR), lambda: (0, 0)),
        out_shape=jax.ShapeDtypeStruct((_LR, _LR), jnp.float32),
    )(t9, bp.reshape(1, 1))

    g4 = g3.reshape(32, _LR // (_BR // 4), _BR // 4, _LR)
    res = pl.pallas_call(
        _b_body,
        grid=(_HR // _BR,),
        in_specs=[
            pl.BlockSpec((_LR, _LR), lambda i: (0, 0)),
            pl.BlockSpec((3, _BR, _HR), lambda i: (0, i, 0)),
            pl.BlockSpec((32, 1, _BR // 4, _LR), lambda i: (0, i, 0, 0)),
            pl.BlockSpec((32, 4), lambda i: (0, 0)),
            pl.BlockSpec((32, 1), lambda i: (0, 0)),
            pl.BlockSpec((1, 32), lambda i: (0, 0)),
            pl.BlockSpec((1, 1), lambda i: (0, 0)),
        ],
        out_specs=pl.BlockSpec((_BR, _HR), lambda i: (i, 0)),
        out_shape=jax.ShapeDtypeStruct((_HR, _HR), jnp.float32),
        scratch_shapes=[
            pltpu.SMEM((1, 1), jnp.float32),
            pltpu.VMEM((_BR, _HR), jnp.float32),
            pltpu.VMEM((_BR, _HR), jnp.float32),
            pltpu.VMEM((32, _BR // 4, _HR), jnp.float32),
        ],
    )(p, hr, g4, w4t, bs1.reshape(32, 1), jnp.transpose(Ws2),
      bs2.reshape(1, 1))

    return res.reshape(1, 1, _HR, _HR)


# speedup vs baseline: 1.8444x; 1.8444x over previous
import jax
import jax.numpy as jnp
from jax.experimental import pallas as pl
from jax.experimental.pallas import tpu as pltpu

_HR = 2048
_LR = 512
_KP = 262144.0
_DIL = 15
_BR = 16
_EX = _BR + 14


def _a1_body(x_ref, w1_ref, b1_ref, wsc_ref, wp9_ref, g_ref, t9_ref):
    x = x_ref[...].reshape(27, 128 * _LR)
    f = jnp.maximum(
        jnp.dot(w1_ref[...], x, preferred_element_type=jnp.float32)
        + b1_ref[...], 0.0)
    g = jnp.dot(wsc_ref[...], f, preferred_element_type=jnp.float32)
    t9 = jnp.dot(wp9_ref[...], f, preferred_element_type=jnp.float32)
    g_ref[...] = g.reshape(32, 128, _LR)
    t9_ref[...] = t9.reshape(9, 128, _LR)


def _a2_body(t9_ref, bp_ref, p_ref):
    acc = jnp.zeros((_LR, _LR), jnp.float32) + bp_ref[0, 0]
    t9 = t9_ref[...]
    for ky in range(3):
        for kx in range(3):
            dy, dx = ky - 1, kx - 1
            a, b = max(0, -dy), _LR - max(0, dy)
            c, d = max(0, -dx), _LR - max(0, dx)
            sl = t9[ky * 3 + kx, a + dy:b + dy, c + dx:d + dx]
            acc = acc + jnp.pad(sl, ((a, _LR - b), (c, _LR - d)))
    p_ref[...] = jax.nn.sigmoid(acc)


def _b_body(p_ref, hr_ref, g_ref, w4_ref, bs1_ref, ws2_ref, bs2_ref,
            o_ref, carry_ref):
    i = pl.program_id(0)

    @pl.when(i == 0)
    def _():
        carry_ref[0, 0] = 0.0

    j = _BR * i - 7 + jax.lax.broadcasted_iota(jnp.int32, (_EX, 1), 0)
    jc = jnp.clip(j, 0, _HR - 1)
    f = (jc.astype(jnp.float32) + 0.5) * 0.25 - 0.5
    a = jnp.floor(f)
    w = f - a
    ai = a.astype(jnp.int32)
    a0 = jnp.clip(ai, 0, _LR - 1)
    a1 = jnp.clip(ai + 1, 0, _LR - 1)
    lanes = jax.lax.broadcasted_iota(jnp.int32, (_EX, _LR), 1)
    vmat = (jnp.where(lanes == a0, 1.0 - w, 0.0)
            + jnp.where(lanes == a1, w, 0.0))
    upv = jnp.dot(vmat, p_ref[...], preferred_element_type=jnp.float32)

    prv = jnp.concatenate([upv[:, :1], upv[:, :-1]], axis=1)
    nxt = jnp.concatenate([upv[:, 1:], upv[:, -1:]], axis=1)
    r4 = jax.lax.broadcasted_iota(jnp.int32, (1, _HR), 1) % 4
    wprv = jnp.where(r4 == 0, 0.375, jnp.where(r4 == 1, 0.125, 0.0))
    wcur = jnp.where((r4 == 0) | (r4 == 3), 0.625, 0.875)
    wnxt = jnp.where(r4 == 2, 0.125, jnp.where(r4 == 3, 0.375, 0.0))
    up_ext = (jnp.repeat(prv, 4, axis=1) * wprv
              + jnp.repeat(upv, 4, axis=1) * wcur
              + jnp.repeat(nxt, 4, axis=1) * wnxt)

    valid = ((j >= 0) & (j < _HR)).astype(jnp.float32)
    m_ext = ((up_ext > 0.01) & (up_ext < 0.99)).astype(jnp.float32) * valid

    dv = m_ext[0:_BR]
    for s in range(1, _DIL):
        dv = jnp.maximum(dv, m_ext[s:s + _BR])
    ph = jnp.concatenate(
        [jnp.zeros((_BR, 7), jnp.float32), dv, jnp.zeros((_BR, 7), jnp.float32)],
        axis=1)
    dil = ph[:, 0:_HR]
    for s in range(1, _DIL):
        dil = jnp.maximum(dil, ph[:, s:s + _HR])

    rowsum = jnp.sum(dil, axis=1, keepdims=True)
    q = jax.lax.broadcasted_iota(jnp.int32, (_BR, _BR), 0)
    r = jax.lax.broadcasted_iota(jnp.int32, (_BR, _BR), 1)
    row_excl = jnp.dot((r < q).astype(jnp.float32), rowsum,
                       preferred_element_type=jnp.float32)
    m3 = dil.reshape(_BR, 16, 128)
    csums = jnp.sum(m3, axis=-1)
    k16 = jax.lax.broadcasted_iota(jnp.int32, (16, 16), 0)
    c16 = jax.lax.broadcasted_iota(jnp.int32, (16, 16), 1)
    chunk_excl = jnp.dot(csums, (k16 < c16).astype(jnp.float32),
                         preferred_element_type=jnp.float32)
    i128 = jax.lax.broadcasted_iota(jnp.int32, (128, 128), 0)
    l128 = jax.lax.broadcasted_iota(jnp.int32, (128, 128), 1)
    within = jnp.dot(m3.reshape(_BR * 16, 128),
                     (i128 <= l128).astype(jnp.float32),
                     preferred_element_type=jnp.float32).reshape(_BR, 16, 128)
    incl = (within + chunk_excl[:, :, None] + row_excl[:, :, None]
            ).reshape(_BR, _HR) + carry_ref[0, 0]
    refined = (dil > 0.0) & (incl <= _KP)
    carry_ref[0, 0] += jnp.sum(rowsum)

    up_blk = up_ext[7:7 + _BR]
    norm = (up_blk - 0.5) * 2.0
    x4 = jnp.concatenate([hr_ref[...], norm[None]], axis=0)
    x4f = x4.reshape(4, _BR * _HR)
    g = g_ref[...].reshape(32, _BR // 4, _LR)
    parts = []
    for qq in range(_BR // 4):
        geq = jnp.repeat(g[:, qq, :], 4, axis=-1)
        parts.extend([geq, geq, geq, geq])
    ctxf = jnp.concatenate(parts, axis=1)
    h1 = jnp.maximum(
        jnp.dot(w4_ref[...], x4f, preferred_element_type=jnp.float32)
        + ctxf + bs1_ref[...], 0.0)
    out = jax.nn.sigmoid(
        jnp.dot(ws2_ref[...], h1, preferred_element_type=jnp.float32)
        + bs2_ref[0, 0]).reshape(_BR, _HR)

    o_ref[...] = jnp.where(refined, out, up_blk)


def kernel(lr_image, hr_image, pos, W1, b1, Wp, bp, Ws1, bs1, Ws2, bs2):
    lr = lr_image.reshape(3, _LR, _LR)
    hr = hr_image.reshape(3, _HR, _HR)

    lrp = jnp.pad(lr, ((0, 0), (1, 1), (1, 1)))
    x27 = jnp.stack(
        [lrp[c, ky:ky + _LR, kx:kx + _LR]
         for c in range(3) for ky in range(3) for kx in range(3)], axis=0)

    w1m = W1.reshape(32, 27)
    wp9 = jnp.transpose(Wp.reshape(32, 9))
    wsct = jnp.transpose(Ws1[4:])
    w4t = jnp.transpose(Ws1[:4])

    g3, t9 = pl.pallas_call(
        _a1_body,
        grid=(4,),
        in_specs=[
            pl.BlockSpec((27, 128, _LR), lambda i: (0, i, 0)),
            pl.BlockSpec((32, 27), lambda i: (0, 0)),
            pl.BlockSpec((32, 1), lambda i: (0, 0)),
            pl.BlockSpec((32, 32), lambda i: (0, 0)),
            pl.BlockSpec((9, 32), lambda i: (0, 0)),
        ],
        out_specs=[
            pl.BlockSpec((32, 128, _LR), lambda i: (0, i, 0)),
            pl.BlockSpec((9, 128, _LR), lambda i: (0, i, 0)),
        ],
        out_shape=[
            jax.ShapeDtypeStruct((32, _LR, _LR), jnp.float32),
            jax.ShapeDtypeStruct((9, _LR, _LR), jnp.float32),
        ],
    )(x27, w1m, b1.reshape(32, 1), wsct, wp9)

    p = pl.pallas_call(
        _a2_body,
        in_specs=[
            pl.BlockSpec((9, _LR, _LR), lambda: (0, 0, 0)),
            pl.BlockSpec((1, 1), lambda: (0, 0)),
        ],
        out_specs=pl.BlockSpec((_LR, _LR), lambda: (0, 0)),
        out_shape=jax.ShapeDtypeStruct((_LR, _LR), jnp.float32),
    )(t9, bp.reshape(1, 1))

    g4 = g3.reshape(32, _LR // (_BR // 4), _BR // 4, _LR)
    res = pl.pallas_call(
        _b_body,
        grid=(_HR // _BR,),
        in_specs=[
            pl.BlockSpec((_LR, _LR), lambda i: (0, 0)),
            pl.BlockSpec((3, _BR, _HR), lambda i: (0, i, 0)),
            pl.BlockSpec((32, 1, _BR // 4, _LR), lambda i: (0, i, 0, 0)),
            pl.BlockSpec((32, 4), lambda i: (0, 0)),
            pl.BlockSpec((32, 1), lambda i: (0, 0)),
            pl.BlockSpec((1, 32), lambda i: (0, 0)),
            pl.BlockSpec((1, 1), lambda i: (0, 0)),
        ],
        out_specs=pl.BlockSpec((_BR, _HR), lambda i: (i, 0)),
        out_shape=jax.ShapeDtypeStruct((_HR, _HR), jnp.float32),
        scratch_shapes=[pltpu.SMEM((1, 1), jnp.float32)],
    )(p, hr, g4, w4t, bs1.reshape(32, 1), jnp.transpose(Ws2),
      bs2.reshape(1, 1))

    return res.reshape(1, 1, _HR, _HR)
